# baseline (speedup 1.0000x reference)
import functools

import jax
import jax.numpy as jnp
from jax import lax
from jax.experimental import pallas as pl
from jax.experimental.pallas import tpu as pltpu
from jax.experimental.pallas import tpu_sc as plsc

R, C = 16384, 200
V, D = 2000, 9
N = R * C
L = 16
NC, NS = 2, 16
NW = NC * NS
COLS = R // NW
SLAB = COLS // 2
KS = SLAB // L
TBL_PAD = 18048
V_PAD = 2048
RS_ITERS = V // L


def _sc_partial_sums(movies_t, table_flat):
    mesh = plsc.VectorSubcoreMesh(core_axis_name="c", subcore_axis_name="s")

    @functools.partial(
        pl.kernel, mesh=mesh,
        out_type=jax.ShapeDtypeStruct((NW, L), jnp.float32),
        compiler_params=pltpu.CompilerParams(needs_layout_passes=False),
        scratch_types=[
            pltpu.VMEM((C, SLAB), jnp.int32),
            pltpu.VMEM((C, SLAB), jnp.int32),
            pltpu.VMEM((TBL_PAD,), jnp.float32),
            pltpu.VMEM((V_PAD,), jnp.float32),
            pltpu.VMEM((V_PAD,), jnp.float32),
            pltpu.VMEM((L,), jnp.float32),
            pltpu.SemaphoreType.DMA,
            pltpu.SemaphoreType.DMA,
        ],
    )
    def k(mov_hbm, tbl_hbm, out_hbm, mov_a, mov_b, tbl_v, rowsum_v, count_v,
          acc_v, sem_a, sem_b):
        wid = lax.axis_index("s") * NC + lax.axis_index("c")
        base = wid * COLS
        pltpu.sync_copy(tbl_hbm, tbl_v.at[pl.ds(0, V * D)])
        h_a = pltpu.async_copy(
            mov_hbm.at[:, pl.ds(base, SLAB)], mov_a, sem_a)

        zero = jnp.zeros((L,), jnp.float32)
        ones = jnp.ones((L,), jnp.float32)
        lane9 = lax.iota(jnp.int32, L) * D

        def init_body(b, _):
            count_v[pl.ds(b * L, L)] = zero
            return 0

        lax.fori_loop(0, V_PAD // L, init_body, 0)

        def rs_body(b, _):
            flat_base = b * (L * D)
            acc = plsc.load_gather(tbl_v, [lane9 + flat_base])
            for kk in range(1, D):
                acc = acc + plsc.load_gather(tbl_v, [lane9 + (flat_base + kk)])
            rowsum_v[pl.ds(b * L, L)] = acc
            return 0

        lax.fori_loop(0, RS_ITERS, rs_body, 0)

        h_a.wait()
        h_b = pltpu.async_copy(
            mov_hbm.at[:, pl.ds(base + SLAB, SLAB)], mov_b, sem_b)

        def count_chunk(mov_v):
            def body(r, _):
                for kk in range(KS):
                    idx = mov_v[r, pl.ds(kk * L, L)]
                    plsc.addupdate_scatter(count_v, [idx], ones)
                return 0
            lax.fori_loop(0, C, body, 0)

        count_chunk(mov_a)
        h_b.wait()
        count_chunk(mov_b)

        def dot_body(b, acc):
            cnt = count_v[pl.ds(b * L, L)]
            rs = rowsum_v[pl.ds(b * L, L)]
            return acc + cnt * rs

        acc_v[...] = lax.fori_loop(0, RS_ITERS, dot_body, zero)
        pltpu.sync_copy(acc_v, out_hbm.at[wid])

    return k(movies_t, table_flat)


def _tc_finish(partials):
    def body(p_ref, o_ref):
        o_ref[0, 0] = jax.nn.sigmoid(jnp.sum(p_ref[...]) * (1.0 / (N * D)))

    return pl.pallas_call(
        body,
        out_shape=jax.ShapeDtypeStruct((1, 1), jnp.float32),
        out_specs=pl.BlockSpec(memory_space=pltpu.SMEM),
    )(partials)


def kernel(movies, ratings, table):
    del ratings
    partials = _sc_partial_sums(movies.T, table.reshape(-1))
    return _tc_finish(partials)[0, 0]


# speedup vs baseline: 1.8265x; 1.8265x over previous
import functools

import jax
import jax.numpy as jnp
from jax import lax
from jax.experimental import pallas as pl
from jax.experimental.pallas import tpu as pltpu
from jax.experimental.pallas import tpu_sc as plsc

R, C = 16384, 200
V, D = 2000, 9
N = R * C
L = 16
NC, NS = 2, 16
NW = NC * NS
COLS = R // NW
SLAB = COLS // 2
KS = SLAB // L
RU = 2
TBL_PAD = 18048
V_PAD = 2048
RS_ITERS = V // L


def _sc_partial_sums(movies_t, table_flat):
    mesh = plsc.VectorSubcoreMesh(core_axis_name="c", subcore_axis_name="s")

    @functools.partial(
        pl.kernel, mesh=mesh,
        out_type=jax.ShapeDtypeStruct((NW, L), jnp.float32),
        compiler_params=pltpu.CompilerParams(needs_layout_passes=False),
        scratch_types=[
            pltpu.VMEM((C, SLAB), jnp.int32),
            pltpu.VMEM((C, SLAB), jnp.int32),
            pltpu.VMEM((TBL_PAD,), jnp.float32),
            pltpu.VMEM((V_PAD,), jnp.float32),
            pltpu.VMEM((L,), jnp.float32),
            pltpu.SemaphoreType.DMA,
            pltpu.SemaphoreType.DMA,
        ],
    )
    def k(mov_hbm, tbl_hbm, out_hbm, mov_a, mov_b, tbl_v, rowsum_v, acc_v,
          sem_a, sem_b):
        wid = lax.axis_index("s") * NC + lax.axis_index("c")
        base = wid * COLS
        pltpu.sync_copy(tbl_hbm, tbl_v.at[pl.ds(0, V * D)])
        h_a = pltpu.async_copy(
            mov_hbm.at[:, pl.ds(base, SLAB)], mov_a, sem_a)

        lane9 = lax.iota(jnp.int32, L) * D

        def rs_body(b, _):
            flat_base = b * (L * D)
            acc = plsc.load_gather(tbl_v, [lane9 + flat_base])
            for kk in range(1, D):
                acc = acc + plsc.load_gather(tbl_v, [lane9 + (flat_base + kk)])
            rowsum_v[pl.ds(b * L, L)] = acc
            return 0

        lax.fori_loop(0, RS_ITERS, rs_body, 0)

        h_a.wait()
        h_b = pltpu.async_copy(
            mov_hbm.at[:, pl.ds(base + SLAB, SLAB)], mov_b, sem_b)

        def gather_chunk(mov_v, accs):
            def body(i, accs):
                accs = list(accs)
                for ru in range(RU):
                    r = i * RU + ru
                    for kk in range(KS):
                        idx = mov_v[r, pl.ds(kk * L, L)]
                        g = plsc.load_gather(rowsum_v, [idx])
                        slot = (ru * KS + kk) % 4
                        accs[slot] = accs[slot] + g
                return tuple(accs)
            return lax.fori_loop(0, C // RU, body, accs)

        zero = jnp.zeros((L,), jnp.float32)
        accs = gather_chunk(mov_a, (zero,) * 4)
        h_b.wait()
        a0, a1, a2, a3 = gather_chunk(mov_b, accs)
        acc_v[...] = (a0 + a1) + (a2 + a3)
        pltpu.sync_copy(acc_v, out_hbm.at[wid])

    return k(movies_t, table_flat)


def _tc_finish(partials):
    def body(p_ref, o_ref):
        o_ref[0, 0] = jax.nn.sigmoid(jnp.sum(p_ref[...]) * (1.0 / (N * D)))

    return pl.pallas_call(
        body,
        out_shape=jax.ShapeDtypeStruct((1, 1), jnp.float32),
        out_specs=pl.BlockSpec(memory_space=pltpu.SMEM),
    )(partials)


def kernel(movies, ratings, table):
    del ratings
    partials = _sc_partial_sums(movies.T, table.reshape(-1))
    return _tc_finish(partials)[0, 0]


# speedup vs baseline: 1.8347x; 1.0045x over previous
import functools

import jax
import jax.numpy as jnp
from jax import lax
from jax.experimental import pallas as pl
from jax.experimental.pallas import tpu as pltpu
from jax.experimental.pallas import tpu_sc as plsc

R, C = 16384, 200
V, D = 2000, 9
N = R * C
L = 16
NC, NS = 2, 16
NW = NC * NS
COLS = R // NW
SLAB = COLS // 2
KS = SLAB // L
RU = 1
TBL_PAD = 18048
V_PAD = 2048
RS_ITERS = V // L


def _sc_partial_sums(movies_t, table_flat):
    mesh = plsc.VectorSubcoreMesh(core_axis_name="c", subcore_axis_name="s")

    @functools.partial(
        pl.kernel, mesh=mesh,
        out_type=jax.ShapeDtypeStruct((NW, L), jnp.float32),
        compiler_params=pltpu.CompilerParams(needs_layout_passes=False),
        scratch_types=[
            pltpu.VMEM((C, SLAB), jnp.int32),
            pltpu.VMEM((C, SLAB), jnp.int32),
            pltpu.VMEM((TBL_PAD,), jnp.float32),
            pltpu.VMEM((V_PAD,), jnp.float32),
            pltpu.VMEM((L,), jnp.float32),
            pltpu.SemaphoreType.DMA,
            pltpu.SemaphoreType.DMA,
        ],
    )
    def k(mov_hbm, tbl_hbm, out_hbm, mov_a, mov_b, tbl_v, rowsum_v, acc_v,
          sem_a, sem_b):
        wid = lax.axis_index("s") * NC + lax.axis_index("c")
        base = wid * COLS
        pltpu.sync_copy(tbl_hbm, tbl_v.at[pl.ds(0, V * D)])
        h_a = pltpu.async_copy(
            mov_hbm.at[:, pl.ds(base, SLAB)], mov_a, sem_a)

        lane9 = lax.iota(jnp.int32, L) * D

        def rs_body(b, _):
            flat_base = b * (L * D)
            acc = plsc.load_gather(tbl_v, [lane9 + flat_base])
            for kk in range(1, D):
                acc = acc + plsc.load_gather(tbl_v, [lane9 + (flat_base + kk)])
            rowsum_v[pl.ds(b * L, L)] = acc
            return 0

        lax.fori_loop(0, RS_ITERS, rs_body, 0)

        h_a.wait()
        h_b = pltpu.async_copy(
            mov_hbm.at[:, pl.ds(base + SLAB, SLAB)], mov_b, sem_b)

        def gather_chunk(mov_v, accs):
            def body(i, accs):
                accs = list(accs)
                for ru in range(RU):
                    r = i * RU + ru
                    for kk in range(KS):
                        idx = mov_v[r, pl.ds(kk * L, L)]
                        g = plsc.load_gather(rowsum_v, [idx])
                        slot = (ru * KS + kk) % 4
                        accs[slot] = accs[slot] + g
                return tuple(accs)
            return lax.fori_loop(0, C // RU, body, accs)

        zero = jnp.zeros((L,), jnp.float32)
        accs = gather_chunk(mov_a, (zero,) * 4)
        h_b.wait()
        a0, a1, a2, a3 = gather_chunk(mov_b, accs)
        acc_v[...] = (a0 + a1) + (a2 + a3)
        pltpu.sync_copy(acc_v, out_hbm.at[wid])

    return k(movies_t, table_flat)


def _tc_finish(partials):
    def body(p_ref, o_ref):
        o_ref[0, 0] = jax.nn.sigmoid(jnp.sum(p_ref[...]) * (1.0 / (N * D)))

    return pl.pallas_call(
        body,
        out_shape=jax.ShapeDtypeStruct((1, 1), jnp.float32),
        out_specs=pl.BlockSpec(memory_space=pltpu.SMEM),
    )(partials)


def kernel(movies, ratings, table):
    del ratings
    partials = _sc_partial_sums(movies.T, table.reshape(-1))
    return _tc_finish(partials)[0, 0]


# speedup vs baseline: 1.8398x; 1.0028x over previous
import functools

import jax
import jax.numpy as jnp
from jax import lax
from jax.experimental import pallas as pl
from jax.experimental.pallas import tpu as pltpu
from jax.experimental.pallas import tpu_sc as plsc

R, C = 16384, 200
V, D = 2000, 9
N = R * C
L = 16
NC, NS = 2, 16
NW = NC * NS
COLS = R // NW
SLAB = COLS // 2
KS = SLAB // L
RU = 1
TBL_PAD = 18048
V_PAD = 2048
RS_ITERS = V // L


def _sc_partial_sums(movies_t, table_flat):
    mesh = plsc.VectorSubcoreMesh(core_axis_name="c", subcore_axis_name="s")

    @functools.partial(
        pl.kernel, mesh=mesh,
        out_type=jax.ShapeDtypeStruct((NW, L), jnp.float32),
        compiler_params=pltpu.CompilerParams(needs_layout_passes=False),
        scratch_types=[
            pltpu.VMEM((C, SLAB), jnp.int32),
            pltpu.VMEM((C, SLAB), jnp.int32),
            pltpu.VMEM((TBL_PAD,), jnp.float32),
            pltpu.VMEM((V_PAD,), jnp.float32),
            pltpu.VMEM((L,), jnp.float32),
            pltpu.SemaphoreType.DMA,
            pltpu.SemaphoreType.DMA,
        ],
    )
    def k(mov_hbm, tbl_hbm, out_hbm, mov_a, mov_b, tbl_v, rowsum_v, acc_v,
          sem_a, sem_b):
        wid = lax.axis_index("s") * NC + lax.axis_index("c")
        base = wid * COLS
        pltpu.sync_copy(tbl_hbm, tbl_v.at[pl.ds(0, V * D)])
        h_a = pltpu.async_copy(
            mov_hbm.at[:, pl.ds(base, SLAB)], mov_a, sem_a)

        lane9 = lax.iota(jnp.int32, L) * D

        def rs_body(b, _):
            flat_base = b * (L * D)
            acc = plsc.load_gather(tbl_v, [lane9 + flat_base])
            for kk in range(1, D):
                acc = acc + plsc.load_gather(tbl_v, [lane9 + (flat_base + kk)])
            rowsum_v[pl.ds(b * L, L)] = acc
            return 0

        lax.fori_loop(0, RS_ITERS, rs_body, 0)

        h_a.wait()
        h_b = pltpu.async_copy(
            mov_hbm.at[:, pl.ds(base + SLAB, SLAB)], mov_b, sem_b)

        def gather_chunk(mov_v, accs):
            def body(i, accs):
                accs = list(accs)
                for ru in range(RU):
                    r = i * RU + ru
                    for kk in range(KS):
                        idx = mov_v[r, pl.ds(kk * L, L)]
                        g = plsc.load_gather(rowsum_v, [idx])
                        slot = (ru * KS + kk) % 8
                        accs[slot] = accs[slot] + g
                return tuple(accs)
            return lax.fori_loop(0, C // RU, body, accs)

        zero = jnp.zeros((L,), jnp.float32)
        accs = gather_chunk(mov_a, (zero,) * 8)
        h_b.wait()
        accs = gather_chunk(mov_b, accs)
        acc_v[...] = sum(accs[1:], accs[0])
        pltpu.sync_copy(acc_v, out_hbm.at[wid])

    return k(movies_t, table_flat)


def _tc_finish(partials):
    def body(p_ref, o_ref):
        o_ref[0, 0] = jax.nn.sigmoid(jnp.sum(p_ref[...]) * (1.0 / (N * D)))

    return pl.pallas_call(
        body,
        out_shape=jax.ShapeDtypeStruct((1, 1), jnp.float32),
        out_specs=pl.BlockSpec(memory_space=pltpu.SMEM),
    )(partials)


def kernel(movies, ratings, table):
    del ratings
    partials = _sc_partial_sums(movies.T, table.reshape(-1))
    return _tc_finish(partials)[0, 0]
